# baseline (speedup 1.0000x reference)
import functools

import jax
import jax.numpy as jnp
from jax import lax
from jax.experimental import pallas as pl
from jax.experimental.pallas import tpu as pltpu
from jax.experimental.pallas import tpu_sc as plsc

_H = 12
_K = 8
_TAU_EPS = 1e-06
_T_SCALAR = 1.0
_DT = 1.0

_PREC = lax.Precision.HIGHEST
_PREC_REF = lax.Precision.DEFAULT

_NW = 32
_CHUNK = 64


def _matmul_kern(x_ref, w_ref, b_ref, o_ref):
    o_ref[...] = (
        jnp.dot(x_ref[...], w_ref[...], preferred_element_type=jnp.float32,
                precision=_PREC_REF)
        + b_ref[...]
    )


def _matmul(x, w, b, blk_m=512):
    m, kdim = x.shape
    n = w.shape[1]
    grid = (m // blk_m,)
    return pl.pallas_call(
        _matmul_kern,
        grid=grid,
        in_specs=[
            pl.BlockSpec((blk_m, kdim), lambda i: (i, 0)),
            pl.BlockSpec((kdim, n), lambda i: (0, 0)),
            pl.BlockSpec((1, n), lambda i: (0, 0)),
        ],
        out_specs=pl.BlockSpec((blk_m, n), lambda i: (i, 0)),
        out_shape=jax.ShapeDtypeStruct((m, n), jnp.float32),
    )(x, w, b)


def _select_kern(q_ref, kt_ref, o_ref):
    q = q_ref[0, 0]
    kt = kt_ref[0, 0]
    tq = q.shape[0]
    t = kt.shape[1]
    bh = pl.program_id(0)

    s = jnp.dot(q, kt, preferred_element_type=jnp.float32, precision=_PREC_REF)

    col = lax.broadcasted_iota(jnp.int32, (tq, t), 1)
    neg_inf = jnp.float32(-jnp.inf)
    cols = []
    for _ in range(_K):
        idx = jnp.argmax(s, axis=1).astype(jnp.int32)[:, None]
        s = jnp.where(col == idx, neg_inf, s)
        cols.append(idx)
    o_ref[...] = jnp.concatenate(cols, axis=1) + bh * t


def _select(qh, khT, blk_q=256):
    b, h, t, dh = qh.shape
    n = b * h * t
    grid = (b * h, t // blk_q)
    return pl.pallas_call(
        _select_kern,
        grid=grid,
        in_specs=[
            pl.BlockSpec((1, 1, blk_q, dh), lambda bh, qi: (bh // _H, bh % _H, qi, 0)),
            pl.BlockSpec((1, 1, dh, t), lambda bh, qi: (bh // _H, bh % _H, 0, 0)),
        ],
        out_specs=pl.BlockSpec((blk_q, _K),
                               lambda bh, qi: (bh * (t // blk_q) + qi, 0)),
        out_shape=jax.ShapeDtypeStruct((n, _K), jnp.int32),
    )(qh, khT)


def _sc_attend(gidx, vt, fkt, fqt, t):
    n = vt.shape[0]
    dh = vt.shape[1]
    per_tile = n // _NW
    n_chunks = per_tile // _CHUNK
    mesh = plsc.VectorSubcoreMesh(core_axis_name="c", subcore_axis_name="s")

    @functools.partial(
        pl.kernel, mesh=mesh,
        compiler_params=pltpu.CompilerParams(needs_layout_passes=False,
                                             use_tc_tiling_on_sc=False),
        out_type=jax.ShapeDtypeStruct((n, dh), jnp.float32),
        scratch_types=[
            pltpu.VMEM((_CHUNK * _K,), jnp.int32),
            pltpu.VMEM((_CHUNK * _K,), jnp.int32),
            pltpu.VMEM((_CHUNK * _K, dh), jnp.float32),
            pltpu.VMEM((_CHUNK * _K, dh), jnp.float32),
            pltpu.VMEM((2 * t * 4,), jnp.float32),
            pltpu.VMEM((4, n // _NW), jnp.float32),
            pltpu.VMEM((_CHUNK, dh), jnp.float32),
            pltpu.SemaphoreType.DMA,
            pltpu.SemaphoreType.DMA,
        ],
    )
    def sck(gidx_hbm, vt_hbm, fkt_hbm, fqt_hbm, out_hbm,
            idx_v0, idx_v1, rows_v0, rows_v1, fk_v, fqt_v, out_v, sem0, sem1):
        wid = lax.axis_index("s") * 2 + lax.axis_index("c")
        base = wid * per_tile
        lane = lax.iota(jnp.int32, 16)
        idx_bufs = (idx_v0, idx_v1)
        row_bufs = (rows_v0, rows_v1)
        sems = (sem0, sem1)

        grp_lo = base // t
        hi_needed = (base + per_tile - 1) // t != grp_lo
        pltpu.sync_copy(fkt_hbm.at[pl.ds(grp_lo * t * 4, t * 4)],
                        fk_v.at[pl.ds(0, t * 4)])
        @pl.when(hi_needed)
        def _():
            pltpu.sync_copy(fkt_hbm.at[pl.ds((grp_lo + 1) * t * 4, t * 4)],
                            fk_v.at[pl.ds(t * 4, t * 4)])
        pltpu.sync_copy(fqt_hbm.at[:, pl.ds(base, per_tile)], fqt_v)

        def start_chunk(ci, par):
            cbase = base + ci * _CHUNK
            pltpu.sync_copy(gidx_hbm.at[pl.ds(cbase * _K, _CHUNK * _K)],
                            idx_bufs[par])
            pltpu.make_async_copy(vt_hbm.at[idx_bufs[par]], row_bufs[par],
                                  sems[par]).start()

        start_chunk(0, 0)
        start_chunk(1, 1)

        def compute_chunk(ci, par):
            idx_v = idx_bufs[par]
            rows_v = row_bufs[par]
            cbase = base + ci * _CHUNK
            coff = ci * _CHUNK
            pltpu.make_async_copy(vt_hbm.at[idx_v], rows_v, sems[par]).wait()

            def group_body(g, c3):
                tok = g * 16 + lane
                qphi = fqt_v[0, pl.ds(coff + g * 16, 16)]
                qta = fqt_v[1, pl.ds(coff + g * 16, 16)]
                qtb = fqt_v[2, pl.ds(coff + g * 16, 16)]
                qtau = fqt_v[3, pl.ds(coff + g * 16, 16)]
                ls = []
                for j in range(_K):
                    gI = plsc.load_gather(idx_v, [tok * _K + j])
                    loc4 = (gI - grp_lo * t) * 4
                    fphi = plsc.load_gather(fk_v, [loc4])
                    fta = plsc.load_gather(fk_v, [loc4 + 1])
                    ftb = plsc.load_gather(fk_v, [loc4 + 2])
                    ftau = plsc.load_gather(fk_v, [loc4 + 3])
                    u = qphi + fphi
                    z = jnp.exp(-jnp.abs(u))
                    phi = jnp.where(u >= 0.0, 1.0 / (1.0 + z), z / (1.0 + z))
                    c = (qta + fta) * _T_SCALAR + (qtb + ftb)
                    z2 = jnp.exp(-jnp.abs(c))
                    tint = jnp.where(c >= 0.0, 1.0 / (1.0 + z2), z2 / (1.0 + z2))
                    u2 = qtau + ftau
                    z3 = jnp.exp(-jnp.abs(u2))
                    l1p = z3 * (0.999998767 + z3 * (-0.499872009 + z3 * (
                        0.331121163 + z3 * (-0.235150498 + z3 * (
                            0.149437105 + z3 * (-0.066589603 + z3 * 0.014203161))))))
                    tau = jnp.maximum(u2, 0.0) + l1p + _TAU_EPS
                    logit = phi / tau * (1.0 - jnp.exp(-tau * tint))
                    ls.append(logit)
                mx = ls[0]
                for j in range(1, _K):
                    mx = jnp.maximum(mx, ls[j])
                es = [jnp.exp(l - mx) for l in ls]
                ssum = es[0]
                for j in range(1, _K):
                    ssum = ssum + es[j]
                inv = _DT / ssum
                ws = [es[j] * inv for j in range(_K)]
                for tl in range(16):
                    t8 = (g * 16 + tl) * _K
                    for seg in range(dh // 16):
                        acc = rows_v[t8, pl.ds(seg * 16, 16)] * ws[0][tl]
                        for j in range(1, _K):
                            acc = acc + rows_v[t8 + j, pl.ds(seg * 16, 16)] * ws[j][tl]
                        out_v[g * 16 + tl, pl.ds(seg * 16, 16)] = acc

            def group2_body(gg, c4):
                group_body(gg * 2, 0)
                group_body(gg * 2 + 1, 0)
                return c4

            lax.fori_loop(0, _CHUNK // 32, group2_body, 0)
            pltpu.sync_copy(out_v, out_hbm.at[pl.ds(cbase, _CHUNK), :])

        def pair_body(cp, carry):
            for par in range(2):
                ci = cp * 2 + par
                compute_chunk(ci, par)
                @pl.when(ci + 2 < n_chunks)
                def _():
                    start_chunk(ci + 2, par)
            return carry

        lax.fori_loop(0, n_chunks // 2, pair_body, 0)

    return sck(gidx, vt, fkt, fqt)


def kernel(x, Wq, bq, Wk, bk, Wv, bv, Wo, bo, Wta, bta, Wtb, btb,
           Wpi, bpi, Wpo, bpo, Wti, bti, Wto, bto):
    b, t, d = x.shape
    h = _H
    dh = d // h
    n = b * h * t

    w_phi = (Wpi @ Wpo)[:, 0]
    b_phi = bpi @ Wpo[:, 0] + bpo[0]
    w_tau = (Wti @ Wto)[:, 0]
    b_tau = bti @ Wto[:, 0] + bto[0]
    w_ta = Wta[:, 0]
    w_tb = Wtb[:, 0]
    feats_q = [w_phi[:dh], w_ta[:dh], w_tb[:dh], w_tau[:dh]]
    feats_k = [w_phi[dh:], w_ta[dh:], w_tb[dh:], w_tau[dh:]]
    eye_h = jnp.eye(h, dtype=jnp.float32)
    sq = jnp.stack([jnp.kron(eye_h, f[:, None]) for f in feats_q], axis=-1)
    sq = sq.reshape(d, 4 * h)
    sk = jnp.stack([jnp.kron(eye_h, f[:, None]) for f in feats_k], axis=-1)
    sk = sk.reshape(d, 4 * h)
    fbias = jnp.array([b_phi, bta[0], btb[0], b_tau], jnp.float32)

    pad = 32
    wbig = jnp.concatenate(
        [Wq, Wk, Wv, Wq @ sq, Wk @ sk,
         jnp.zeros((d, pad), jnp.float32)], axis=1)
    bbig = jnp.concatenate(
        [bq, bk, bv, bq @ sq + jnp.tile(fbias, h), bk @ sk,
         jnp.zeros((pad,), jnp.float32)])[None, :]
    y = _matmul(x.reshape(b * t, d), wbig, bbig)

    qh = y[:, :d].reshape(b, t, h, dh).transpose(0, 2, 1, 3)
    khT = y[:, d:2 * d].reshape(b, t, h, dh).transpose(0, 2, 3, 1)
    vt = y[:, 2 * d:3 * d].reshape(b, t, h, dh).transpose(0, 2, 1, 3).reshape(n, dh)
    fqt = y[:, 3 * d:3 * d + 4 * h].reshape(b, t, h, 4).transpose(0, 2, 1, 3)
    fqt = fqt.reshape(n, 4).T
    fkt = y[:, 3 * d + 4 * h:3 * d + 8 * h].reshape(b, t, h, 4)
    fkt = fkt.transpose(0, 2, 1, 3).reshape(n * 4)

    gidx = _select(qh, khT).reshape(n * _K)

    out = _sc_attend(gidx, vt, fkt, fqt, t)

    combined = out.reshape(b, h, t, dh).transpose(0, 2, 1, 3).reshape(b * t, d)
    yo = _matmul(combined, Wo, bo[None, :])
    return yo.reshape(b, t, d)


# speedup vs baseline: 1.0641x; 1.0641x over previous
import functools

import jax
import jax.numpy as jnp
from jax import lax
from jax.experimental import pallas as pl
from jax.experimental.pallas import tpu as pltpu
from jax.experimental.pallas import tpu_sc as plsc

_H = 12
_K = 8
_TAU_EPS = 1e-06
_T_SCALAR = 1.0
_DT = 1.0

_PREC = lax.Precision.HIGHEST
_PREC_REF = lax.Precision.DEFAULT

_NW = 32
_CHUNK = 64


def _matmul_kern(x_ref, w_ref, b_ref, o_ref):
    o_ref[...] = (
        jnp.dot(x_ref[...], w_ref[...], preferred_element_type=jnp.float32,
                precision=_PREC_REF)
        + b_ref[...]
    )


def _matmul(x, w, b, blk_m=512):
    m, kdim = x.shape
    n = w.shape[1]
    grid = (m // blk_m,)
    return pl.pallas_call(
        _matmul_kern,
        grid=grid,
        in_specs=[
            pl.BlockSpec((blk_m, kdim), lambda i: (i, 0)),
            pl.BlockSpec((kdim, n), lambda i: (0, 0)),
            pl.BlockSpec((1, n), lambda i: (0, 0)),
        ],
        out_specs=pl.BlockSpec((blk_m, n), lambda i: (i, 0)),
        out_shape=jax.ShapeDtypeStruct((m, n), jnp.float32),
    )(x, w, b)


def _select_kern(q_ref, kt_ref, o_ref):
    q = q_ref[0, 0]
    kt = kt_ref[0, 0]
    tq = q.shape[0]
    t = kt.shape[1]
    bh = pl.program_id(0)

    s = jnp.dot(q, kt, preferred_element_type=jnp.float32, precision=_PREC_REF)

    col = lax.broadcasted_iota(jnp.int32, (tq, t), 1)
    neg_inf = jnp.float32(-jnp.inf)
    cols = []
    for _ in range(_K):
        idx = jnp.argmax(s, axis=1).astype(jnp.int32)[:, None]
        s = jnp.where(col == idx, neg_inf, s)
        cols.append(idx)
    o_ref[...] = jnp.concatenate(cols, axis=1) + bh * t


def _select(qh, khT, blk_q=256):
    b, h, t, dh = qh.shape
    n = b * h * t
    grid = (b * h, t // blk_q)
    return pl.pallas_call(
        _select_kern,
        grid=grid,
        in_specs=[
            pl.BlockSpec((1, 1, blk_q, dh), lambda bh, qi: (bh // _H, bh % _H, qi, 0)),
            pl.BlockSpec((1, 1, dh, t), lambda bh, qi: (bh // _H, bh % _H, 0, 0)),
        ],
        out_specs=pl.BlockSpec((blk_q, _K),
                               lambda bh, qi: (bh * (t // blk_q) + qi, 0)),
        out_shape=jax.ShapeDtypeStruct((n, _K), jnp.int32),
    )(qh, khT)


def _sc_attend(gidx, vt, fkt, fqt, t):
    n = vt.shape[0]
    dh = vt.shape[1]
    per_tile = n // _NW
    n_chunks = per_tile // _CHUNK
    mesh = plsc.VectorSubcoreMesh(core_axis_name="c", subcore_axis_name="s")

    @functools.partial(
        pl.kernel, mesh=mesh,
        compiler_params=pltpu.CompilerParams(needs_layout_passes=False,
                                             use_tc_tiling_on_sc=False),
        out_type=jax.ShapeDtypeStruct((n, dh), jnp.float32),
        scratch_types=[
            pltpu.VMEM((_CHUNK * _K,), jnp.int32),
            pltpu.VMEM((_CHUNK * _K,), jnp.int32),
            pltpu.VMEM((_CHUNK * _K, dh), jnp.float32),
            pltpu.VMEM((_CHUNK * _K, dh), jnp.float32),
            pltpu.VMEM((2 * t * 4,), jnp.float32),
            pltpu.VMEM((4, n // _NW), jnp.float32),
            pltpu.VMEM((_CHUNK, dh), jnp.float32),
            pltpu.SemaphoreType.DMA,
            pltpu.SemaphoreType.DMA,
        ],
    )
    def sck(gidx_hbm, vt_hbm, fkt_hbm, fqt_hbm, out_hbm,
            idx_v0, idx_v1, rows_v0, rows_v1, fk_v, fqt_v, out_v, sem0, sem1):
        wid = lax.axis_index("s") * 2 + lax.axis_index("c")
        base = wid * per_tile
        lane = lax.iota(jnp.int32, 16)
        idx_bufs = (idx_v0, idx_v1)
        row_bufs = (rows_v0, rows_v1)
        sems = (sem0, sem1)

        grp_lo = base // t
        hi_needed = (base + per_tile - 1) // t != grp_lo
        pltpu.sync_copy(fkt_hbm.at[pl.ds(grp_lo * t * 4, t * 4)],
                        fk_v.at[pl.ds(0, t * 4)])
        @pl.when(hi_needed)
        def _():
            pltpu.sync_copy(fkt_hbm.at[pl.ds((grp_lo + 1) * t * 4, t * 4)],
                            fk_v.at[pl.ds(t * 4, t * 4)])
        pltpu.sync_copy(fqt_hbm.at[:, pl.ds(base, per_tile)], fqt_v)

        def start_chunk(ci, par):
            cbase = base + ci * _CHUNK
            pltpu.sync_copy(gidx_hbm.at[pl.ds(cbase * _K, _CHUNK * _K)],
                            idx_bufs[par])
            pltpu.make_async_copy(vt_hbm.at[idx_bufs[par]], row_bufs[par],
                                  sems[par]).start()

        start_chunk(0, 0)
        start_chunk(1, 1)

        def compute_chunk(ci, par):
            idx_v = idx_bufs[par]
            rows_v = row_bufs[par]
            cbase = base + ci * _CHUNK
            coff = ci * _CHUNK
            pltpu.make_async_copy(vt_hbm.at[idx_v], rows_v, sems[par]).wait()

            def group_body(g, c3):
                tok = g * 16 + lane
                qphi = fqt_v[0, pl.ds(coff + g * 16, 16)]
                qta = fqt_v[1, pl.ds(coff + g * 16, 16)]
                qtb = fqt_v[2, pl.ds(coff + g * 16, 16)]
                qtau = fqt_v[3, pl.ds(coff + g * 16, 16)]
                ls = []
                for j in range(_K):
                    gI = plsc.load_gather(idx_v, [tok * _K + j])
                    loc4 = (gI - grp_lo * t) * 4
                    fphi = plsc.load_gather(fk_v, [loc4])
                    fta = plsc.load_gather(fk_v, [loc4 + 1])
                    ftb = plsc.load_gather(fk_v, [loc4 + 2])
                    ftau = plsc.load_gather(fk_v, [loc4 + 3])
                    u = qphi + fphi
                    z = jnp.exp(-jnp.abs(u))
                    phi = jnp.where(u >= 0.0, 1.0 / (1.0 + z), z / (1.0 + z))
                    c = (qta + fta) * _T_SCALAR + (qtb + ftb)
                    z2 = jnp.exp(-jnp.abs(c))
                    tint = jnp.where(c >= 0.0, 1.0 / (1.0 + z2), z2 / (1.0 + z2))
                    u2 = qtau + ftau
                    z3 = jnp.exp(-jnp.abs(u2))
                    l1p = z3 * (0.999998767 + z3 * (-0.499872009 + z3 * (
                        0.331121163 + z3 * (-0.235150498 + z3 * (
                            0.149437105 + z3 * (-0.066589603 + z3 * 0.014203161))))))
                    tau = jnp.maximum(u2, 0.0) + l1p + _TAU_EPS
                    logit = phi / tau * (1.0 - jnp.exp(-tau * tint))
                    ls.append(logit)
                mx = ls[0]
                for j in range(1, _K):
                    mx = jnp.maximum(mx, ls[j])
                es = [jnp.exp(l - mx) for l in ls]
                ssum = es[0]
                for j in range(1, _K):
                    ssum = ssum + es[j]
                inv = _DT / ssum
                ws = [es[j] * inv for j in range(_K)]
                for tl in range(16):
                    t8 = (g * 16 + tl) * _K
                    for seg in range(dh // 16):
                        acc = rows_v[t8, pl.ds(seg * 16, 16)] * ws[0][tl]
                        for j in range(1, _K):
                            acc = acc + rows_v[t8 + j, pl.ds(seg * 16, 16)] * ws[j][tl]
                        out_v[g * 16 + tl, pl.ds(seg * 16, 16)] = acc

            def group2_body(gg, c4):
                group_body(gg * 2, 0)
                group_body(gg * 2 + 1, 0)
                return c4

            lax.fori_loop(0, _CHUNK // 32, group2_body, 0)
            pltpu.sync_copy(out_v, out_hbm.at[pl.ds(cbase, _CHUNK), :])

        def pair_body(cp, carry):
            for par in range(2):
                ci = cp * 2 + par
                compute_chunk(ci, par)
                @pl.when(ci + 2 < n_chunks)
                def _():
                    start_chunk(ci + 2, par)
            return carry

        lax.fori_loop(0, n_chunks // 2, pair_body, 0)

    return sck(gidx, vt, fkt, fqt)


def kernel(x, Wq, bq, Wk, bk, Wv, bv, Wo, bo, Wta, bta, Wtb, btb,
           Wpi, bpi, Wpo, bpo, Wti, bti, Wto, bto):
    b, t, d = x.shape
    h = _H
    dh = d // h
    n = b * h * t

    w_phi = (Wpi @ Wpo)[:, 0]
    b_phi = bpi @ Wpo[:, 0] + bpo[0]
    w_tau = (Wti @ Wto)[:, 0]
    b_tau = bti @ Wto[:, 0] + bto[0]
    w_ta = Wta[:, 0]
    w_tb = Wtb[:, 0]
    feats_q = [w_phi[:dh], w_ta[:dh], w_tb[:dh], w_tau[:dh]]
    feats_k = [w_phi[dh:], w_ta[dh:], w_tb[dh:], w_tau[dh:]]
    eye_h = jnp.eye(h, dtype=jnp.float32)
    sq = jnp.stack([jnp.kron(eye_h, f[:, None]) for f in feats_q], axis=-1)
    sq = sq.reshape(d, 4 * h)
    sk = jnp.stack([jnp.kron(eye_h, f[:, None]) for f in feats_k], axis=-1)
    sk = sk.reshape(d, 4 * h)
    fbias = jnp.array([b_phi, bta[0], btb[0], b_tau], jnp.float32)

    pad = 32
    wbig = jnp.concatenate(
        [Wq, Wk, Wv, Wq @ sq, Wk @ sk,
         jnp.zeros((d, pad), jnp.float32)], axis=1)
    bbig = jnp.concatenate(
        [bq, bk, bv, bq @ sq + jnp.tile(fbias, h), bk @ sk,
         jnp.zeros((pad,), jnp.float32)])[None, :]
    y = _matmul(x.reshape(b * t, d), wbig, bbig)

    qh = y[:, :d].reshape(b, t, h, dh).transpose(0, 2, 1, 3)
    khT = y[:, d:2 * d].reshape(b, t, h, dh).transpose(0, 2, 3, 1)
    vt = y[:, 2 * d:3 * d].reshape(b, t, h, dh).transpose(0, 2, 1, 3).reshape(n, dh)
    fqt = y[:, 3 * d:3 * d + 4 * h].reshape(b, t, h, 4).transpose(0, 2, 1, 3)
    fqt = fqt.reshape(n, 4).T
    fkt = y[:, 3 * d + 4 * h:3 * d + 8 * h].reshape(b, t, h, 4)
    fkt = fkt.transpose(0, 2, 1, 3).reshape(n * 4)

    nb = h * t
    outs = []
    for bi in range(b):
        gidx_i = _select(qh[bi:bi + 1], khT[bi:bi + 1]).reshape(nb * _K)
        out_i = _sc_attend(gidx_i, vt[bi * nb:(bi + 1) * nb],
                           fkt[bi * nb * 4:(bi + 1) * nb * 4],
                           fqt[:, bi * nb:(bi + 1) * nb], t)
        outs.append(out_i)
    out = jnp.concatenate(outs, axis=0)

    combined = out.reshape(b, h, t, dh).transpose(0, 2, 1, 3).reshape(b * t, d)
    yo = _matmul(combined, Wo, bo[None, :])
    return yo.reshape(b, t, d)


# speedup vs baseline: 1.1695x; 1.0991x over previous
import functools

import jax
import jax.numpy as jnp
from jax import lax
from jax.experimental import pallas as pl
from jax.experimental.pallas import tpu as pltpu
from jax.experimental.pallas import tpu_sc as plsc

_H = 12
_K = 8
_TAU_EPS = 1e-06
_T_SCALAR = 1.0
_DT = 1.0

_PREC = lax.Precision.HIGHEST
_PREC_REF = lax.Precision.DEFAULT

_NW = 32
_CHUNK = 64


def _matmul_kern(x_ref, w_ref, b_ref, o_ref):
    o_ref[...] = (
        jnp.dot(x_ref[...], w_ref[...], preferred_element_type=jnp.float32,
                precision=_PREC_REF)
        + b_ref[...]
    )


def _matmul(x, w, b, blk_m=512):
    m, kdim = x.shape
    n = w.shape[1]
    grid = (m // blk_m,)
    return pl.pallas_call(
        _matmul_kern,
        grid=grid,
        in_specs=[
            pl.BlockSpec((blk_m, kdim), lambda i: (i, 0)),
            pl.BlockSpec((kdim, n), lambda i: (0, 0)),
            pl.BlockSpec((1, n), lambda i: (0, 0)),
        ],
        out_specs=pl.BlockSpec((blk_m, n), lambda i: (i, 0)),
        out_shape=jax.ShapeDtypeStruct((m, n), jnp.float32),
    )(x, w, b)


def _select_kern(q_ref, kt_ref, o_ref):
    q = q_ref[0, 0]
    kt = kt_ref[0, 0]
    tq = q.shape[0]
    t = kt.shape[1]
    bh = pl.program_id(0)

    s = jnp.dot(q, kt, preferred_element_type=jnp.float32, precision=_PREC_REF)

    col = lax.broadcasted_iota(jnp.int32, (tq, t), 1)
    neg_inf = jnp.float32(-jnp.inf)
    cols = []
    for ki in range(_K):
        idx = jnp.argmax(s, axis=1).astype(jnp.int32)[:, None]
        if ki + 1 < _K:
            s = jnp.where(col == idx, neg_inf, s)
        cols.append(idx)
    o_ref[...] = jnp.concatenate(cols, axis=1) + bh * t


def _select(qh, khT, blk_q=512):
    b, h, t, dh = qh.shape
    n = b * h * t
    grid = (b * h, t // blk_q)
    return pl.pallas_call(
        _select_kern,
        grid=grid,
        in_specs=[
            pl.BlockSpec((1, 1, blk_q, dh), lambda bh, qi: (bh // _H, bh % _H, qi, 0)),
            pl.BlockSpec((1, 1, dh, t), lambda bh, qi: (bh // _H, bh % _H, 0, 0)),
        ],
        out_specs=pl.BlockSpec((blk_q, _K),
                               lambda bh, qi: (bh * (t // blk_q) + qi, 0)),
        out_shape=jax.ShapeDtypeStruct((n, _K), jnp.int32),
    )(qh, khT)


def _sc_attend(gidx, vt, fkt, fqt, t):
    n = vt.shape[0]
    dh = vt.shape[1]
    per_tile = n // _NW
    n_chunks = per_tile // _CHUNK
    mesh = plsc.VectorSubcoreMesh(core_axis_name="c", subcore_axis_name="s")

    @functools.partial(
        pl.kernel, mesh=mesh,
        compiler_params=pltpu.CompilerParams(needs_layout_passes=False,
                                             use_tc_tiling_on_sc=False),
        out_type=jax.ShapeDtypeStruct((n, dh), jnp.float32),
        scratch_types=[
            pltpu.VMEM((_CHUNK * _K,), jnp.int32),
            pltpu.VMEM((_CHUNK * _K,), jnp.int32),
            pltpu.VMEM((_CHUNK * _K, dh), jnp.float32),
            pltpu.VMEM((_CHUNK * _K, dh), jnp.float32),
            pltpu.VMEM((2 * t * 4,), jnp.float32),
            pltpu.VMEM((4, n // _NW), jnp.float32),
            pltpu.VMEM((_CHUNK, dh), jnp.float32),
            pltpu.SemaphoreType.DMA,
            pltpu.SemaphoreType.DMA,
        ],
    )
    def sck(gidx_hbm, vt_hbm, fkt_hbm, fqt_hbm, out_hbm,
            idx_v0, idx_v1, rows_v0, rows_v1, fk_v, fqt_v, out_v, sem0, sem1):
        wid = lax.axis_index("s") * 2 + lax.axis_index("c")
        base = wid * per_tile
        lane = lax.iota(jnp.int32, 16)
        idx_bufs = (idx_v0, idx_v1)
        row_bufs = (rows_v0, rows_v1)
        sems = (sem0, sem1)

        grp_lo = base // t
        hi_needed = (base + per_tile - 1) // t != grp_lo
        pltpu.sync_copy(fkt_hbm.at[pl.ds(grp_lo * t * 4, t * 4)],
                        fk_v.at[pl.ds(0, t * 4)])
        @pl.when(hi_needed)
        def _():
            pltpu.sync_copy(fkt_hbm.at[pl.ds((grp_lo + 1) * t * 4, t * 4)],
                            fk_v.at[pl.ds(t * 4, t * 4)])
        pltpu.sync_copy(fqt_hbm.at[:, pl.ds(base, per_tile)], fqt_v)

        def start_chunk(ci, par):
            cbase = base + ci * _CHUNK
            pltpu.sync_copy(gidx_hbm.at[pl.ds(cbase * _K, _CHUNK * _K)],
                            idx_bufs[par])
            pltpu.make_async_copy(vt_hbm.at[idx_bufs[par]], row_bufs[par],
                                  sems[par]).start()

        start_chunk(0, 0)
        start_chunk(1, 1)

        def compute_chunk(ci, par):
            idx_v = idx_bufs[par]
            rows_v = row_bufs[par]
            cbase = base + ci * _CHUNK
            coff = ci * _CHUNK
            pltpu.make_async_copy(vt_hbm.at[idx_v], rows_v, sems[par]).wait()

            def group_body(g, c3):
                tok = g * 16 + lane
                qphi = fqt_v[0, pl.ds(coff + g * 16, 16)]
                qta = fqt_v[1, pl.ds(coff + g * 16, 16)]
                qtb = fqt_v[2, pl.ds(coff + g * 16, 16)]
                qtau = fqt_v[3, pl.ds(coff + g * 16, 16)]
                ls = []
                for j in range(_K):
                    gI = plsc.load_gather(idx_v, [tok * _K + j])
                    loc4 = (gI - grp_lo * t) * 4
                    fphi = plsc.load_gather(fk_v, [loc4])
                    fta = plsc.load_gather(fk_v, [loc4 + 1])
                    ftb = plsc.load_gather(fk_v, [loc4 + 2])
                    ftau = plsc.load_gather(fk_v, [loc4 + 3])
                    u = qphi + fphi
                    z = jnp.exp(-jnp.abs(u))
                    phi = jnp.where(u >= 0.0, 1.0 / (1.0 + z), z / (1.0 + z))
                    c = (qta + fta) * _T_SCALAR + (qtb + ftb)
                    z2 = jnp.exp(-jnp.abs(c))
                    tint = jnp.where(c >= 0.0, 1.0 / (1.0 + z2), z2 / (1.0 + z2))
                    u2 = qtau + ftau
                    z3 = jnp.exp(-jnp.abs(u2))
                    l1p = z3 * (0.999998767 + z3 * (-0.499872009 + z3 * (
                        0.331121163 + z3 * (-0.235150498 + z3 * (
                            0.149437105 + z3 * (-0.066589603 + z3 * 0.014203161))))))
                    tau = jnp.maximum(u2, 0.0) + l1p + _TAU_EPS
                    logit = phi / tau * (1.0 - jnp.exp(-tau * tint))
                    ls.append(logit)
                mx = ls[0]
                for j in range(1, _K):
                    mx = jnp.maximum(mx, ls[j])
                es = [jnp.exp(l - mx) for l in ls]
                ssum = es[0]
                for j in range(1, _K):
                    ssum = ssum + es[j]
                inv = _DT / ssum
                ws = [es[j] * inv for j in range(_K)]
                for tl in range(16):
                    t8 = (g * 16 + tl) * _K
                    for seg in range(dh // 16):
                        acc = rows_v[t8, pl.ds(seg * 16, 16)] * ws[0][tl]
                        for j in range(1, _K):
                            acc = acc + rows_v[t8 + j, pl.ds(seg * 16, 16)] * ws[j][tl]
                        out_v[g * 16 + tl, pl.ds(seg * 16, 16)] = acc

            def group2_body(gg, c4):
                group_body(gg * 2, 0)
                group_body(gg * 2 + 1, 0)
                return c4

            lax.fori_loop(0, _CHUNK // 32, group2_body, 0)
            pltpu.sync_copy(out_v, out_hbm.at[pl.ds(cbase, _CHUNK), :])

        def pair_body(cp, carry):
            for par in range(2):
                ci = cp * 2 + par
                compute_chunk(ci, par)
                @pl.when(ci + 2 < n_chunks)
                def _():
                    start_chunk(ci + 2, par)
            return carry

        lax.fori_loop(0, n_chunks // 2, pair_body, 0)

    return sck(gidx, vt, fkt, fqt)


def kernel(x, Wq, bq, Wk, bk, Wv, bv, Wo, bo, Wta, bta, Wtb, btb,
           Wpi, bpi, Wpo, bpo, Wti, bti, Wto, bto):
    b, t, d = x.shape
    h = _H
    dh = d // h
    n = b * h * t

    w_phi = (Wpi @ Wpo)[:, 0]
    b_phi = bpi @ Wpo[:, 0] + bpo[0]
    w_tau = (Wti @ Wto)[:, 0]
    b_tau = bti @ Wto[:, 0] + bto[0]
    w_ta = Wta[:, 0]
    w_tb = Wtb[:, 0]
    feats_q = [w_phi[:dh], w_ta[:dh], w_tb[:dh], w_tau[:dh]]
    feats_k = [w_phi[dh:], w_ta[dh:], w_tb[dh:], w_tau[dh:]]
    eye_h = jnp.eye(h, dtype=jnp.float32)
    sq = jnp.stack([jnp.kron(eye_h, f[:, None]) for f in feats_q], axis=-1)
    sq = sq.reshape(d, 4 * h)
    sk = jnp.stack([jnp.kron(eye_h, f[:, None]) for f in feats_k], axis=-1)
    sk = sk.reshape(d, 4 * h)
    fbias = jnp.array([b_phi, bta[0], btb[0], b_tau], jnp.float32)

    pad = 32
    wbig = jnp.concatenate(
        [Wq, Wk, Wv, Wq @ sq, Wk @ sk,
         jnp.zeros((d, pad), jnp.float32)], axis=1)
    bbig = jnp.concatenate(
        [bq, bk, bv, bq @ sq + jnp.tile(fbias, h), bk @ sk,
         jnp.zeros((pad,), jnp.float32)])[None, :]
    y = _matmul(x.reshape(b * t, d), wbig, bbig)

    qh = y[:, :d].reshape(b, t, h, dh).transpose(0, 2, 1, 3)
    khT = y[:, d:2 * d].reshape(b, t, h, dh).transpose(0, 2, 3, 1)
    vt = y[:, 2 * d:3 * d].reshape(b, t, h, dh).transpose(0, 2, 1, 3).reshape(n, dh)
    fqt = y[:, 3 * d:3 * d + 4 * h].reshape(b, t, h, 4).transpose(0, 2, 1, 3)
    fqt = fqt.reshape(n, 4).T
    fkt = y[:, 3 * d + 4 * h:3 * d + 8 * h].reshape(b, t, h, 4)
    fkt = fkt.transpose(0, 2, 1, 3).reshape(n * 4)

    hs = h // 2
    nb = hs * t
    outs = []
    for bi in range(b):
        for hb in range(2):
            p = bi * 2 + hb
            gidx_i = _select(qh[bi:bi + 1, hb * hs:(hb + 1) * hs],
                             khT[bi:bi + 1, hb * hs:(hb + 1) * hs]).reshape(nb * _K)
            out_i = _sc_attend(gidx_i, vt[p * nb:(p + 1) * nb],
                               fkt[p * nb * 4:(p + 1) * nb * 4],
                               fqt[:, p * nb:(p + 1) * nb], t)
            outs.append(out_i)
    out = jnp.concatenate(outs, axis=0)

    combined = out.reshape(b, h, t, dh).transpose(0, 2, 1, 3).reshape(b * t, d)
    yo = _matmul(combined, Wo, bo[None, :])
    return yo.reshape(b, t, d)
